# baseline (speedup 1.0000x reference)
import jax
import jax.numpy as jnp
from jax import lax
from jax.experimental import pallas as pl
from jax.experimental.pallas import tpu as pltpu
from jax.experimental.pallas import tpu_sc as plsc

_B, _L = 1024, 200
_V, _EMB, _PDIM = 1000000, 32, 32
_NW = 32
_CHUNK = 512
_NFULL = _V // _CHUNK
_TAIL = _V - _NFULL * _CHUNK
_SR = _V // 4
_BQ = _B // 256
_UNITS = (_L // 8) * _BQ


def _relayout_body(wt_hbm, tail_hbm, scratch_hbm, inbuf, outbuf, tinbuf, sem):
    w = lax.axis_index("s") * 2 + lax.axis_index("c")
    iota16 = lax.iota(jnp.int32, 16)

    def xpose_rows(nrows):
        def rowfill(r, carry):
            for q in range(4):
                colidx = jnp.full((16,), r * 4 + q, jnp.int32)
                lo = plsc.load_gather(inbuf, [iota16, colidx])
                hi = plsc.load_gather(inbuf, [iota16 + 16, colidx])
                outbuf[r, pl.ds(q * 32, 16)] = lo
                outbuf[r, pl.ds(q * 32 + 16, 16)] = hi
            return carry

        lax.fori_loop(0, nrows, rowfill, 0)

    def outer(j, carry):
        ch = w + j * _NW

        @pl.when(ch < _NFULL)
        def _():
            v0 = pl.multiple_of(ch * _CHUNK, _CHUNK)
            copies = [
                pltpu.async_copy(
                    wt_hbm.at[pl.ds(e * 8, 8), pl.ds(v0, _CHUNK)],
                    inbuf.at[pl.ds(e * 8, 8)],
                    sem,
                )
                for e in range(4)
            ]
            for cp in copies:
                cp.wait()
            xpose_rows(_CHUNK // 4)
            r0 = pl.multiple_of(ch * (_CHUNK // 4), 8)
            pltpu.sync_copy(outbuf, scratch_hbm.at[pl.ds(r0, _CHUNK // 4)])

        return carry

    lax.fori_loop(0, (_NFULL + _NW - 1) // _NW, outer, 0)

    @pl.when(w == _NW - 1)
    def _():
        pltpu.sync_copy(tail_hbm, tinbuf)

        def rowfill(r, carry):
            for q in range(4):
                colidx = jnp.full((16,), r * 4 + q, jnp.int32)
                lo = plsc.load_gather(tinbuf, [iota16, colidx])
                hi = plsc.load_gather(tinbuf, [iota16 + 16, colidx])
                outbuf[r, pl.ds(q * 32, 16)] = lo
                outbuf[r, pl.ds(q * 32 + 16, 16)] = hi
            return carry

        lax.fori_loop(0, _TAIL // 4, rowfill, 0)
        pltpu.sync_copy(
            outbuf.at[pl.ds(0, _TAIL // 4)],
            scratch_hbm.at[pl.ds(_SR - _TAIL // 4, _TAIL // 4)],
        )


def _lookup_body(
    scratch_hbm, ids_hbm, pos_hbm, out_hbm,
    posv, idsbuf, rowsbuf, colsbuf, wordbuf, plane, sem, gsem,
):
    w = lax.axis_index("s") * 2 + lax.axis_index("c")
    iota16 = lax.iota(jnp.int32, 16)
    pltpu.sync_copy(pos_hbm, posv)

    def do_unit(u):
        lg = u // _BQ
        bq = u % _BQ
        l0 = pl.multiple_of(lg * 8, 8)
        b0 = pl.multiple_of(bq * 256, 128)
        pltpu.sync_copy(ids_hbm.at[pl.ds(l0, 8), pl.ds(b0, 256)], idsbuf)

        def idxfill(i, carry):
            row = i // 16
            col = (i % 16) * 16
            v = idsbuf[row, pl.ds(col, 16)]
            rowsbuf[row, pl.ds(col, 16)] = lax.shift_right_logical(v, 2)
            colsbuf[row, pl.ds(col, 16)] = lax.shift_left(
                lax.bitwise_and(v, 3), 5
            )
            return carry

        lax.fori_loop(0, 8 * 16, idxfill, 0)

        def do_row(l, carry):
            gcs = [
                pltpu.async_copy(
                    scratch_hbm.at[rowsbuf.at[l, pl.ds(c * 128, 128)]],
                    wordbuf.at[pl.ds(c * 128, 128)],
                    gsem,
                )
                for c in range(2)
            ]
            lvec = jnp.full((16,), l0 + l, jnp.int32)
            bvecs = [
                plsc.load_gather(
                    posv, [jnp.full((16,), p, jnp.int32), lvec]
                )
                for p in range(_PDIM)
            ]

            def posfill(tg, c2):
                t0 = tg * 16
                for p in range(_PDIM):
                    plane[_EMB + p, pl.ds(t0, 16)] = bvecs[p]
                return c2

            lax.fori_loop(0, 16, posfill, 0)
            for gc in gcs:
                gc.wait()

            def wfill(tg, c2):
                t0 = tg * 16
                tidx = t0 + iota16
                cbase = colsbuf[l, pl.ds(t0, 16)]
                for f in range(_EMB):
                    vec = plsc.load_gather(wordbuf, [tidx, cbase + f])
                    plane[f, pl.ds(t0, 16)] = vec
                return c2

            lax.fori_loop(0, 16, wfill, 0)
            pltpu.sync_copy(plane, out_hbm.at[l0 + l, :, pl.ds(b0, 256)])
            return carry

        lax.fori_loop(0, 8, do_row, 0)

    for j in range(4):
        u = w + j * _NW

        @pl.when(u < _UNITS)
        def _():
            do_unit(u)


@jax.jit
def _emb_fused(ids_t, wt_t, tail_t, pos_t):
    mesh = plsc.VectorSubcoreMesh(core_axis_name="c", subcore_axis_name="s")
    params = pltpu.CompilerParams(
        use_tc_tiling_on_sc=True, needs_layout_passes=False
    )
    scratch = pl.kernel(
        _relayout_body,
        mesh=mesh,
        compiler_params=params,
        out_type=jax.ShapeDtypeStruct((_SR, 128), jnp.float32),
        scratch_types=[
            pltpu.VMEM((_EMB, _CHUNK), jnp.float32),
            pltpu.VMEM((_CHUNK // 4, 128), jnp.float32),
            pltpu.VMEM((_EMB, _TAIL), jnp.float32),
            pltpu.SemaphoreType.DMA,
        ],
    )(wt_t, tail_t)
    out_t = pl.kernel(
        _lookup_body,
        mesh=mesh,
        compiler_params=params,
        out_type=jax.ShapeDtypeStruct((_L, _EMB + _PDIM, _B), jnp.float32),
        scratch_types=[
            pltpu.VMEM((_PDIM, 512), jnp.float32),
            pltpu.VMEM((8, 256), jnp.int32),
            pltpu.VMEM((8, 256), jnp.int32),
            pltpu.VMEM((8, 256), jnp.int32),
            pltpu.VMEM((256, 128), jnp.float32),
            pltpu.VMEM((_EMB + _PDIM, 256), jnp.float32),
            pltpu.SemaphoreType.DMA,
            pltpu.SemaphoreType.DMA,
        ],
    )(scratch, ids_t, pos_t)
    return out_t


def kernel(input_ids, word_table, pos_table):
    ids_t = input_ids.T
    wt_t = word_table.T
    tail_t = word_table[_NFULL * _CHUNK:, :].T
    pos_t = pos_table.T
    out_t = _emb_fused(ids_t, wt_t, tail_t, pos_t)
    return jnp.transpose(out_t, (2, 0, 1))


# speedup vs baseline: 2.8619x; 2.8619x over previous
import jax
import jax.numpy as jnp
from jax import lax
from jax.experimental import pallas as pl
from jax.experimental.pallas import tpu as pltpu
from jax.experimental.pallas import tpu_sc as plsc

_B, _L = 1024, 200
_V, _EMB, _PDIM = 1000000, 32, 32
_NW = 32
_CHUNK = 512
_NFULL = _V // _CHUNK
_TAIL = _V - _NFULL * _CHUNK
_SR = _V // 4
_BQ = _B // 128
_UNITS = (_L // 8) * _BQ
_JMAX = 61


def _relayout_body(
    wt_hbm, tail_hbm, scratch_hbm,
    inA, inB, outA, outB, tinbuf, semA, semB, wsemA, wsemB,
):
    w = lax.axis_index("s") * 2 + lax.axis_index("c")
    iota16 = lax.iota(jnp.int32, 16)
    idiv4 = lax.shift_right_logical(iota16, 2)
    iand3_32 = lax.shift_left(lax.bitwise_and(iota16, 3), 5)
    rots = [lax.bitwise_and(iota16 + d, 15) for d in range(16)]

    def fire_loads(ch, buf, sem):
        v0 = pl.multiple_of(ch * _CHUNK, _CHUNK)
        return [
            pltpu.async_copy(
                wt_hbm.at[pl.ds(e * 8, 8), pl.ds(v0, _CHUNK)],
                buf.at[pl.ds(e * 8, 8)],
                sem,
            )
            for e in range(4)
        ]

    def xpose(src, dst, ngroups):
        def vgroup(vg, carry):
            vcol = vg * 16 + iota16
            vrow4 = vg * 4 + idiv4
            for f0 in (0, 16):
                for d in range(16):
                    svec = plsc.load_gather(src, [f0 + rots[d], vcol])
                    plsc.store_scatter(
                        dst, [vrow4, iand3_32 + rots[d] + f0], svec
                    )
            return carry

        lax.fori_loop(0, ngroups, vgroup, 0)

    def fire_store(ch, out):
        r0 = pl.multiple_of(ch * (_CHUNK // 4), 8)
        return pltpu.async_copy(
            out, scratch_hbm.at[pl.ds(r0, _CHUNK // 4)],
            wsemA if out is outA else wsemB,
        )

    for cp in fire_loads(w, inA, semA):
        pass

    def body(jj, carry):
        c0 = w + (2 * jj) * _NW
        c1 = c0 + _NW
        c2 = c1 + _NW
        l1 = fire_loads(c1, inB, semB)

        @pl.when(jj > 0)
        def _():
            pltpu.make_async_copy(
                outA, scratch_hbm.at[pl.ds(0, _CHUNK // 4)], wsemA
            ).wait()

        pltpu.make_async_copy(
            wt_hbm.at[pl.ds(0, 8), pl.ds(0, _CHUNK)],
            inA.at[pl.ds(0, 8)], semA,
        ).wait()
        pltpu.make_async_copy(
            wt_hbm.at[pl.ds(0, 8), pl.ds(0, _CHUNK)],
            inA.at[pl.ds(0, 8)], semA,
        ).wait()
        pltpu.make_async_copy(
            wt_hbm.at[pl.ds(0, 8), pl.ds(0, _CHUNK)],
            inA.at[pl.ds(0, 8)], semA,
        ).wait()
        pltpu.make_async_copy(
            wt_hbm.at[pl.ds(0, 8), pl.ds(0, _CHUNK)],
            inA.at[pl.ds(0, 8)], semA,
        ).wait()
        xpose(inA, outA, _CHUNK // 16)
        fire_store(c0, outA)
        fire_loads(c2, inA, semA)

        @pl.when(jj > 0)
        def _():
            pltpu.make_async_copy(
                outB, scratch_hbm.at[pl.ds(0, _CHUNK // 4)], wsemB
            ).wait()

        for cp in l1:
            cp.wait()
        xpose(inB, outB, _CHUNK // 16)
        fire_store(c1, outB)
        return carry

    lax.fori_loop(0, (_JMAX - 1) // 2, body, 0)

    c60 = w + 60 * _NW
    pltpu.make_async_copy(
        outA, scratch_hbm.at[pl.ds(0, _CHUNK // 4)], wsemA
    ).wait()
    for e in range(4):
        pltpu.make_async_copy(
            wt_hbm.at[pl.ds(0, 8), pl.ds(0, _CHUNK)],
            inA.at[pl.ds(0, 8)], semA,
        ).wait()
    xpose(inA, outA, _CHUNK // 16)
    fire_store(c60, outA)
    pltpu.make_async_copy(
        outA, scratch_hbm.at[pl.ds(0, _CHUNK // 4)], wsemA
    ).wait()
    pltpu.make_async_copy(
        outB, scratch_hbm.at[pl.ds(0, _CHUNK // 4)], wsemB
    ).wait()

    @pl.when(w == 0)
    def _():
        ch = _NFULL - 1
        for cp in fire_loads(ch, inA, semA):
            cp.wait()
        xpose(inA, outA, _CHUNK // 16)
        r0 = pl.multiple_of(ch * (_CHUNK // 4), 8)
        pltpu.sync_copy(outA, scratch_hbm.at[pl.ds(r0, _CHUNK // 4)])

    @pl.when(w == _NW - 1)
    def _():
        pltpu.sync_copy(tail_hbm, tinbuf)

        def vgroup(vg, carry):
            vcol = vg * 16 + iota16
            vrow4 = vg * 4 + idiv4
            for f0 in (0, 16):
                for d in range(16):
                    svec = plsc.load_gather(tinbuf, [f0 + rots[d], vcol])
                    plsc.store_scatter(
                        outB, [vrow4, iand3_32 + rots[d] + f0], svec
                    )
            return carry

        lax.fori_loop(0, _TAIL // 16, vgroup, 0)
        pltpu.sync_copy(
            outB.at[pl.ds(0, _TAIL // 4)],
            scratch_hbm.at[pl.ds(_SR - _TAIL // 4, _TAIL // 4)],
        )


def _lookup_body(
    scratch_hbm, ids_hbm, pos_hbm, out_hbm,
    posv, idsbuf, rowsbuf, colsbuf, wbA, wbB, plA, plB,
    sem, gsem, wsemA, wsemB,
):
    w = lax.axis_index("s") * 2 + lax.axis_index("c")
    iota16 = lax.iota(jnp.int32, 16)
    rots = [lax.bitwise_and(iota16 + d, 15) for d in range(16)]
    pltpu.sync_copy(pos_hbm, posv)

    def do_unit(u):
        lg = u // _BQ
        bq = u % _BQ
        l0 = pl.multiple_of(lg * 8, 8)
        b0 = pl.multiple_of(bq * 128, 128)
        pltpu.sync_copy(ids_hbm.at[pl.ds(l0, 8), pl.ds(b0, 128)], idsbuf)

        def idxfill(i, carry):
            row = i // 8
            col = (i % 8) * 16
            v = idsbuf[row, pl.ds(col, 16)]
            rowsbuf[row, pl.ds(col, 16)] = lax.shift_right_logical(v, 2)
            colsbuf[row, pl.ds(col, 16)] = lax.shift_left(
                lax.bitwise_and(v, 3), 5
            )
            return carry

        lax.fori_loop(0, 8 * 8, idxfill, 0)

        def fire_gather(l, wb):
            return pltpu.async_copy(
                scratch_hbm.at[rowsbuf.at[l]], wb, gsem
            )

        def fill_plane(l, plane):
            lvec = jnp.full((16,), l0 + l, jnp.int32)

            def posfill(p, c2):
                bvec = plsc.load_gather(
                    posv, [jnp.full((16,), p, jnp.int32), lvec]
                )
                for tg in range(8):
                    plane[_EMB + p, pl.ds(tg * 16, 16)] = bvec
                return c2

            lax.fori_loop(0, _PDIM, posfill, 0)

        def word_fill(l, wb, plane):
            def wfill(tg, c2):
                t0 = tg * 16
                tvec = t0 + iota16
                cbase = colsbuf[l, pl.ds(t0, 16)]
                for f0 in (0, 16):
                    for d in range(16):
                        fvec = rots[d] + f0
                        svec = plsc.load_gather(wb, [tvec, cbase + fvec])
                        plsc.store_scatter(plane, [fvec, tvec], svec)
                return c2

            lax.fori_loop(0, 8, wfill, 0)

        def drain_gather(wb):
            pltpu.make_async_copy(
                scratch_hbm.at[pl.ds(0, 128)], wb, gsem
            ).wait()

        def drain_write(plane, wsem):
            pltpu.make_async_copy(
                plane, out_hbm.at[0, :, pl.ds(0, 128)], wsem
            ).wait()

        def fire_write(l, plane, wsem):
            return pltpu.async_copy(
                plane, out_hbm.at[l0 + l, :, pl.ds(b0, 128)], wsem
            )

        fire_gather(0, wbA)

        def lpair(ll, carry):
            lA = ll * 2
            lB = lA + 1
            fire_gather(lB, wbB)

            @pl.when(ll > 0)
            def _():
                drain_write(plA, wsemA)

            fill_plane(lA, plA)
            drain_gather(wbA)
            word_fill(lA, wbA, plA)
            fire_write(lA, plA, wsemA)

            @pl.when(ll < 3)
            def _():
                fire_gather(lB + 1, wbA)

            @pl.when(ll > 0)
            def _():
                drain_write(plB, wsemB)

            fill_plane(lB, plB)
            drain_gather(wbB)
            word_fill(lB, wbB, plB)
            fire_write(lB, plB, wsemB)
            return carry

        lax.fori_loop(0, 4, lpair, 0)
        drain_write(plA, wsemA)
        drain_write(plB, wsemB)

    def unit_loop(j, carry):
        u = w + j * _NW

        @pl.when(u < _UNITS)
        def _():
            do_unit(u)

        return carry

    lax.fori_loop(0, 7, unit_loop, 0)


@jax.jit
def _emb_fused(ids_t, wt_t, tail_t, pos_t):
    mesh = plsc.VectorSubcoreMesh(core_axis_name="c", subcore_axis_name="s")
    params = pltpu.CompilerParams(
        use_tc_tiling_on_sc=True, needs_layout_passes=False
    )
    scratch = pl.kernel(
        _relayout_body,
        mesh=mesh,
        compiler_params=params,
        out_type=jax.ShapeDtypeStruct((_SR, 128), jnp.float32),
        scratch_types=[
            pltpu.VMEM((_EMB, _CHUNK), jnp.float32),
            pltpu.VMEM((_EMB, _CHUNK), jnp.float32),
            pltpu.VMEM((_CHUNK // 4, 128), jnp.float32),
            pltpu.VMEM((_CHUNK // 4, 128), jnp.float32),
            pltpu.VMEM((_EMB, _TAIL), jnp.float32),
            pltpu.SemaphoreType.DMA,
            pltpu.SemaphoreType.DMA,
            pltpu.SemaphoreType.DMA,
            pltpu.SemaphoreType.DMA,
        ],
    )(wt_t, tail_t)
    out_t = pl.kernel(
        _lookup_body,
        mesh=mesh,
        compiler_params=params,
        out_type=jax.ShapeDtypeStruct((_L, _EMB + _PDIM, _B), jnp.float32),
        scratch_types=[
            pltpu.VMEM((_PDIM, 512), jnp.float32),
            pltpu.VMEM((8, 128), jnp.int32),
            pltpu.VMEM((8, 128), jnp.int32),
            pltpu.VMEM((8, 128), jnp.int32),
            pltpu.VMEM((128, 128), jnp.float32),
            pltpu.VMEM((128, 128), jnp.float32),
            pltpu.VMEM((_EMB + _PDIM, 128), jnp.float32),
            pltpu.VMEM((_EMB + _PDIM, 128), jnp.float32),
            pltpu.SemaphoreType.DMA,
            pltpu.SemaphoreType.DMA,
            pltpu.SemaphoreType.DMA,
            pltpu.SemaphoreType.DMA,
        ],
    )(scratch, ids_t, pos_t)
    return out_t


def kernel(input_ids, word_table, pos_table):
    ids_t = input_ids.T
    wt_t = word_table.T
    tail_t = word_table[_NFULL * _CHUNK:, :].T
    pos_t = pos_table.T
    out_t = _emb_fused(ids_t, wt_t, tail_t, pos_t)
    return jnp.transpose(out_t, (2, 0, 1))


# speedup vs baseline: 2.8698x; 1.0027x over previous
import jax
import jax.numpy as jnp
from jax import lax
from jax.experimental import pallas as pl
from jax.experimental.pallas import tpu as pltpu
from jax.experimental.pallas import tpu_sc as plsc

_B, _L = 1024, 200
_V, _EMB, _PDIM = 1000000, 32, 32
_NW = 32
_CHUNK = 512
_NFULL = _V // _CHUNK
_TAIL = _V - _NFULL * _CHUNK
_SR = _V // 4
_BQ = _B // 128
_UNITS = (_L // 8) * _BQ
_JMAX = 61


def _relayout_body(
    wt_hbm, tail_hbm, scratch_hbm,
    inA, inB, outA, outB, tinbuf, semA, semB, wsemA, wsemB,
):
    w = lax.axis_index("s") * 2 + lax.axis_index("c")
    iota16 = lax.iota(jnp.int32, 16)
    idiv4 = lax.shift_right_logical(iota16, 2)
    iand3_32 = lax.shift_left(lax.bitwise_and(iota16, 3), 5)
    rots = [lax.bitwise_and(iota16 + d, 15) for d in range(16)]
    cols_d = [iand3_32 + rots[d] for d in range(16)]

    def fire_loads(ch, buf, sem):
        v0 = pl.multiple_of(ch * _CHUNK, _CHUNK)
        return [
            pltpu.async_copy(
                wt_hbm.at[:, pl.ds(v0, _CHUNK)], buf, sem
            )
        ]

    def xpose(src, dst, ngroups):
        def vgroup(vg, carry):
            vcol = vg * 16 + iota16
            vrow4 = vg * 4 + idiv4
            for f0 in (0, 16):
                for d in range(16):
                    svec = plsc.load_gather(src, [rots[d] + f0, vcol])
                    plsc.store_scatter(dst, [vrow4, cols_d[d] + f0], svec)
            return carry

        lax.fori_loop(0, ngroups, vgroup, 0)

    def fire_store(ch, out):
        r0 = pl.multiple_of(ch * (_CHUNK // 4), 8)
        return pltpu.async_copy(
            out, scratch_hbm.at[pl.ds(r0, _CHUNK // 4)],
            wsemA if out is outA else wsemB,
        )

    for cp in fire_loads(w, inA, semA):
        pass

    def body(jj, carry):
        c0 = w + (2 * jj) * _NW
        c1 = c0 + _NW
        c2 = c1 + _NW
        l1 = fire_loads(c1, inB, semB)

        @pl.when(jj > 0)
        def _():
            pltpu.make_async_copy(
                outA, scratch_hbm.at[pl.ds(0, _CHUNK // 4)], wsemA
            ).wait()

        pltpu.make_async_copy(
            wt_hbm.at[:, pl.ds(0, _CHUNK)], inA, semA
        ).wait()
        xpose(inA, outA, _CHUNK // 16)
        fire_store(c0, outA)
        fire_loads(c2, inA, semA)

        @pl.when(jj > 0)
        def _():
            pltpu.make_async_copy(
                outB, scratch_hbm.at[pl.ds(0, _CHUNK // 4)], wsemB
            ).wait()

        for cp in l1:
            cp.wait()
        xpose(inB, outB, _CHUNK // 16)
        fire_store(c1, outB)
        return carry

    lax.fori_loop(0, (_JMAX - 1) // 2, body, 0)

    c60 = w + 60 * _NW
    pltpu.make_async_copy(
        outA, scratch_hbm.at[pl.ds(0, _CHUNK // 4)], wsemA
    ).wait()
    pltpu.make_async_copy(
        wt_hbm.at[:, pl.ds(0, _CHUNK)], inA, semA
    ).wait()
    xpose(inA, outA, _CHUNK // 16)
    fire_store(c60, outA)
    pltpu.make_async_copy(
        outA, scratch_hbm.at[pl.ds(0, _CHUNK // 4)], wsemA
    ).wait()
    pltpu.make_async_copy(
        outB, scratch_hbm.at[pl.ds(0, _CHUNK // 4)], wsemB
    ).wait()

    @pl.when(w == 0)
    def _():
        ch = _NFULL - 1
        for cp in fire_loads(ch, inA, semA):
            cp.wait()
        xpose(inA, outA, _CHUNK // 16)
        r0 = pl.multiple_of(ch * (_CHUNK // 4), 8)
        pltpu.sync_copy(outA, scratch_hbm.at[pl.ds(r0, _CHUNK // 4)])

    @pl.when(w == _NW - 1)
    def _():
        pltpu.sync_copy(tail_hbm, tinbuf)

        def vgroup(vg, carry):
            vcol = vg * 16 + iota16
            vrow4 = vg * 4 + idiv4
            for f0 in (0, 16):
                for d in range(16):
                    svec = plsc.load_gather(tinbuf, [f0 + rots[d], vcol])
                    plsc.store_scatter(
                        outB, [vrow4, iand3_32 + rots[d] + f0], svec
                    )
            return carry

        lax.fori_loop(0, _TAIL // 16, vgroup, 0)
        pltpu.sync_copy(
            outB.at[pl.ds(0, _TAIL // 4)],
            scratch_hbm.at[pl.ds(_SR - _TAIL // 4, _TAIL // 4)],
        )


def _lookup_body(
    scratch_hbm, ids_hbm, pos_hbm, out_hbm,
    posv, idsbuf, rowsbuf, colsbuf, wbA, wbB, plA, plB,
    sem, gsem, wsemA, wsemB,
):
    w = lax.axis_index("s") * 2 + lax.axis_index("c")
    iota16 = lax.iota(jnp.int32, 16)
    rots = [lax.bitwise_and(iota16 + d, 15) for d in range(16)]
    pltpu.sync_copy(pos_hbm, posv)

    def do_unit(u):
        lg = u // _BQ
        bq = u % _BQ
        l0 = pl.multiple_of(lg * 8, 8)
        b0 = pl.multiple_of(bq * 128, 128)
        pltpu.sync_copy(ids_hbm.at[pl.ds(l0, 8), pl.ds(b0, 128)], idsbuf)

        def idxfill(i, carry):
            row = i // 8
            col = (i % 8) * 16
            v = idsbuf[row, pl.ds(col, 16)]
            rowsbuf[row, pl.ds(col, 16)] = lax.shift_right_logical(v, 2)
            colsbuf[row, pl.ds(col, 16)] = lax.shift_left(
                lax.bitwise_and(v, 3), 5
            )
            return carry

        lax.fori_loop(0, 8 * 8, idxfill, 0)

        def fire_gather(l, wb):
            return pltpu.async_copy(
                scratch_hbm.at[rowsbuf.at[l]], wb, gsem
            )

        def fill_plane(l, plane):
            lvec = jnp.full((16,), l0 + l, jnp.int32)

            def posfill(p, c2):
                bvec = plsc.load_gather(
                    posv, [jnp.full((16,), p, jnp.int32), lvec]
                )
                for tg in range(8):
                    plane[_EMB + p, pl.ds(tg * 16, 16)] = bvec
                return c2

            lax.fori_loop(0, _PDIM, posfill, 0)

        def word_fill(l, wb, plane):
            def wfill(tg, c2):
                t0 = tg * 16
                tvec = t0 + iota16
                cbase = colsbuf[l, pl.ds(t0, 16)]
                for f0 in (0, 16):
                    for d in range(16):
                        fvec = rots[d] + f0
                        svec = plsc.load_gather(wb, [tvec, cbase + fvec])
                        plsc.store_scatter(plane, [fvec, tvec], svec)
                return c2

            lax.fori_loop(0, 8, wfill, 0)

        def drain_gather(wb):
            pltpu.make_async_copy(
                scratch_hbm.at[pl.ds(0, 128)], wb, gsem
            ).wait()

        def drain_write(plane, wsem):
            pltpu.make_async_copy(
                plane, out_hbm.at[0, :, pl.ds(0, 128)], wsem
            ).wait()

        def fire_write(l, plane, wsem):
            return pltpu.async_copy(
                plane, out_hbm.at[l0 + l, :, pl.ds(b0, 128)], wsem
            )

        fire_gather(0, wbA)

        def lpair(ll, carry):
            lA = ll * 2
            lB = lA + 1
            fire_gather(lB, wbB)

            @pl.when(ll > 0)
            def _():
                drain_write(plA, wsemA)

            fill_plane(lA, plA)
            drain_gather(wbA)
            word_fill(lA, wbA, plA)
            fire_write(lA, plA, wsemA)

            @pl.when(ll < 3)
            def _():
                fire_gather(lB + 1, wbA)

            @pl.when(ll > 0)
            def _():
                drain_write(plB, wsemB)

            fill_plane(lB, plB)
            drain_gather(wbB)
            word_fill(lB, wbB, plB)
            fire_write(lB, plB, wsemB)
            return carry

        lax.fori_loop(0, 4, lpair, 0)
        drain_write(plA, wsemA)
        drain_write(plB, wsemB)

    def unit_loop(j, carry):
        u = w + j * _NW

        @pl.when(u < _UNITS)
        def _():
            do_unit(u)

        return carry

    lax.fori_loop(0, 7, unit_loop, 0)


@jax.jit
def _emb_fused(ids_t, wt_t, tail_t, pos_t):
    mesh = plsc.VectorSubcoreMesh(core_axis_name="c", subcore_axis_name="s")
    params = pltpu.CompilerParams(
        use_tc_tiling_on_sc=True, needs_layout_passes=False
    )
    scratch = pl.kernel(
        _relayout_body,
        mesh=mesh,
        compiler_params=params,
        out_type=jax.ShapeDtypeStruct((_SR, 128), jnp.float32),
        scratch_types=[
            pltpu.VMEM((_EMB, _CHUNK), jnp.float32),
            pltpu.VMEM((_EMB, _CHUNK), jnp.float32),
            pltpu.VMEM((_CHUNK // 4, 128), jnp.float32),
            pltpu.VMEM((_CHUNK // 4, 128), jnp.float32),
            pltpu.VMEM((_EMB, _TAIL), jnp.float32),
            pltpu.SemaphoreType.DMA,
            pltpu.SemaphoreType.DMA,
            pltpu.SemaphoreType.DMA,
            pltpu.SemaphoreType.DMA,
        ],
    )(wt_t, tail_t)
    out_t = pl.kernel(
        _lookup_body,
        mesh=mesh,
        compiler_params=params,
        out_type=jax.ShapeDtypeStruct((_L, _EMB + _PDIM, _B), jnp.float32),
        scratch_types=[
            pltpu.VMEM((_PDIM, 512), jnp.float32),
            pltpu.VMEM((8, 128), jnp.int32),
            pltpu.VMEM((8, 128), jnp.int32),
            pltpu.VMEM((8, 128), jnp.int32),
            pltpu.VMEM((128, 128), jnp.float32),
            pltpu.VMEM((128, 128), jnp.float32),
            pltpu.VMEM((_EMB + _PDIM, 128), jnp.float32),
            pltpu.VMEM((_EMB + _PDIM, 128), jnp.float32),
            pltpu.SemaphoreType.DMA,
            pltpu.SemaphoreType.DMA,
            pltpu.SemaphoreType.DMA,
            pltpu.SemaphoreType.DMA,
        ],
    )(scratch, ids_t, pos_t)
    return out_t


def kernel(input_ids, word_table, pos_table):
    ids_t = input_ids.T
    wt_t = word_table.T
    tail_t = word_table[_NFULL * _CHUNK:, :].T
    pos_t = pos_table.T
    out_t = _emb_fused(ids_t, wt_t, tail_t, pos_t)
    return jnp.transpose(out_t, (2, 0, 1))


# speedup vs baseline: 4.6610x; 1.6242x over previous
import jax
import jax.numpy as jnp
from jax import lax
from jax.experimental import pallas as pl
from jax.experimental.pallas import tpu as pltpu
from jax.experimental.pallas import tpu_sc as plsc

_B, _L = 1024, 200
_V, _EMB, _PDIM = 1000000, 32, 32
_NW = 32
_CHUNK = 512
_NFULL = _V // _CHUNK
_TAIL = _V - _NFULL * _CHUNK
_SR = _V // 4
_BQ = _B // 128
_UNITS = (_L // 8) * _BQ
_JMAX = 61


def _relayout_body(
    wt_hbm, tail_hbm, scratch_hbm,
    inA, inB, outA, outB, tinbuf, semA, semB, wsemA, wsemB,
):
    w = lax.axis_index("s") * 2 + lax.axis_index("c")
    iota16 = lax.iota(jnp.int32, 16)
    idiv4 = lax.shift_right_logical(iota16, 2)
    iand3_32 = lax.shift_left(lax.bitwise_and(iota16, 3), 5)
    rots = [lax.bitwise_and(iota16 + d, 15) for d in range(16)]
    cols_d = [iand3_32 + rots[d] for d in range(16)]

    def fire_loads(ch, buf, sem):
        v0 = pl.multiple_of(ch * _CHUNK, _CHUNK)
        return [
            pltpu.async_copy(
                wt_hbm.at[:, pl.ds(v0, _CHUNK)], buf, sem
            )
        ]

    def xpose(src, dst, ngroups):
        @plsc.parallel_loop(0, ngroups, unroll=2)
        def vgroup(vg):
            vcol = vg * 16 + iota16
            vrow4 = vg * 4 + idiv4
            for f0 in (0, 16):
                for d in range(16):
                    svec = plsc.load_gather(src, [rots[d] + f0, vcol])
                    plsc.store_scatter(dst, [vrow4, cols_d[d] + f0], svec)

    def fire_store(ch, out):
        r0 = pl.multiple_of(ch * (_CHUNK // 4), 8)
        return pltpu.async_copy(
            out, scratch_hbm.at[pl.ds(r0, _CHUNK // 4)],
            wsemA if out is outA else wsemB,
        )

    for cp in fire_loads(w, inA, semA):
        pass

    def body(jj, carry):
        c0 = w + (2 * jj) * _NW
        c1 = c0 + _NW
        c2 = c1 + _NW
        l1 = fire_loads(c1, inB, semB)

        @pl.when(jj > 0)
        def _():
            pltpu.make_async_copy(
                outA, scratch_hbm.at[pl.ds(0, _CHUNK // 4)], wsemA
            ).wait()

        pltpu.make_async_copy(
            wt_hbm.at[:, pl.ds(0, _CHUNK)], inA, semA
        ).wait()
        xpose(inA, outA, _CHUNK // 16)
        fire_store(c0, outA)
        fire_loads(c2, inA, semA)

        @pl.when(jj > 0)
        def _():
            pltpu.make_async_copy(
                outB, scratch_hbm.at[pl.ds(0, _CHUNK // 4)], wsemB
            ).wait()

        for cp in l1:
            cp.wait()
        xpose(inB, outB, _CHUNK // 16)
        fire_store(c1, outB)
        return carry

    lax.fori_loop(0, (_JMAX - 1) // 2, body, 0)

    c60 = w + 60 * _NW
    pltpu.make_async_copy(
        outA, scratch_hbm.at[pl.ds(0, _CHUNK // 4)], wsemA
    ).wait()
    pltpu.make_async_copy(
        wt_hbm.at[:, pl.ds(0, _CHUNK)], inA, semA
    ).wait()
    xpose(inA, outA, _CHUNK // 16)
    fire_store(c60, outA)
    pltpu.make_async_copy(
        outA, scratch_hbm.at[pl.ds(0, _CHUNK // 4)], wsemA
    ).wait()
    pltpu.make_async_copy(
        outB, scratch_hbm.at[pl.ds(0, _CHUNK // 4)], wsemB
    ).wait()

    @pl.when(w == 0)
    def _():
        ch = _NFULL - 1
        for cp in fire_loads(ch, inA, semA):
            cp.wait()
        xpose(inA, outA, _CHUNK // 16)
        r0 = pl.multiple_of(ch * (_CHUNK // 4), 8)
        pltpu.sync_copy(outA, scratch_hbm.at[pl.ds(r0, _CHUNK // 4)])

    @pl.when(w == _NW - 1)
    def _():
        pltpu.sync_copy(tail_hbm, tinbuf)

        def vgroup(vg, carry):
            vcol = vg * 16 + iota16
            vrow4 = vg * 4 + idiv4
            for f0 in (0, 16):
                for d in range(16):
                    svec = plsc.load_gather(tinbuf, [f0 + rots[d], vcol])
                    plsc.store_scatter(
                        outB, [vrow4, iand3_32 + rots[d] + f0], svec
                    )
            return carry

        lax.fori_loop(0, _TAIL // 16, vgroup, 0)
        pltpu.sync_copy(
            outB.at[pl.ds(0, _TAIL // 4)],
            scratch_hbm.at[pl.ds(_SR - _TAIL // 4, _TAIL // 4)],
        )


def _lookup_body(
    scratch_hbm, ids_hbm, pos_hbm, out_hbm,
    posv, idsbuf, rowsbuf, colsbuf, wbA, wbB, plA, plB,
    sem, gsem, wsemA, wsemB,
):
    w = lax.axis_index("s") * 2 + lax.axis_index("c")
    iota16 = lax.iota(jnp.int32, 16)
    rots = [lax.bitwise_and(iota16 + d, 15) for d in range(16)]
    pltpu.sync_copy(pos_hbm, posv)

    def do_unit(u):
        lg = u // _BQ
        bq = u % _BQ
        l0 = pl.multiple_of(lg * 8, 8)
        b0 = pl.multiple_of(bq * 128, 128)
        pltpu.sync_copy(ids_hbm.at[pl.ds(l0, 8), pl.ds(b0, 128)], idsbuf)

        def idxfill(i, carry):
            row = i // 8
            col = (i % 8) * 16
            v = idsbuf[row, pl.ds(col, 16)]
            rowsbuf[row, pl.ds(col, 16)] = lax.shift_right_logical(v, 2)
            colsbuf[row, pl.ds(col, 16)] = lax.shift_left(
                lax.bitwise_and(v, 3), 5
            )
            return carry

        lax.fori_loop(0, 8 * 8, idxfill, 0)

        def fire_gather(l, wb):
            return pltpu.async_copy(
                scratch_hbm.at[rowsbuf.at[l]], wb, gsem
            )

        def fill_plane(l, plane):
            lvec = jnp.full((16,), l0 + l, jnp.int32)

            def posfill(p, c2):
                bvec = plsc.load_gather(
                    posv, [jnp.full((16,), p, jnp.int32), lvec]
                )
                for tg in range(8):
                    plane[_EMB + p, pl.ds(tg * 16, 16)] = bvec
                return c2

            lax.fori_loop(0, _PDIM, posfill, 0)

        def word_fill(l, wb, plane):
            def wfill(tg, c2):
                t0 = tg * 16
                tvec = t0 + iota16
                cbase = colsbuf[l, pl.ds(t0, 16)]
                for f0 in (0, 16):
                    for d in range(16):
                        fvec = rots[d] + f0
                        svec = plsc.load_gather(wb, [tvec, cbase + fvec])
                        plsc.store_scatter(plane, [fvec, tvec], svec)
                return c2

            lax.fori_loop(0, 8, wfill, 0)

        def drain_gather(wb):
            pltpu.make_async_copy(
                scratch_hbm.at[pl.ds(0, 128)], wb, gsem
            ).wait()

        def drain_write(plane, wsem):
            pltpu.make_async_copy(
                plane, out_hbm.at[0, :, pl.ds(0, 128)], wsem
            ).wait()

        def fire_write(l, plane, wsem):
            return pltpu.async_copy(
                plane, out_hbm.at[l0 + l, :, pl.ds(b0, 128)], wsem
            )

        fire_gather(0, wbA)

        def lpair(ll, carry):
            lA = ll * 2
            lB = lA + 1
            fire_gather(lB, wbB)

            @pl.when(ll > 0)
            def _():
                drain_write(plA, wsemA)

            fill_plane(lA, plA)
            drain_gather(wbA)
            word_fill(lA, wbA, plA)
            fire_write(lA, plA, wsemA)

            @pl.when(ll < 3)
            def _():
                fire_gather(lB + 1, wbA)

            @pl.when(ll > 0)
            def _():
                drain_write(plB, wsemB)

            fill_plane(lB, plB)
            drain_gather(wbB)
            word_fill(lB, wbB, plB)
            fire_write(lB, plB, wsemB)
            return carry

        lax.fori_loop(0, 4, lpair, 0)
        drain_write(plA, wsemA)
        drain_write(plB, wsemB)

    def unit_loop(j, carry):
        u = w + j * _NW

        @pl.when(u < _UNITS)
        def _():
            do_unit(u)

        return carry

    lax.fori_loop(0, 7, unit_loop, 0)


@jax.jit
def _emb_fused(ids_t, wt_t, tail_t, pos_t):
    mesh = plsc.VectorSubcoreMesh(core_axis_name="c", subcore_axis_name="s")
    params = pltpu.CompilerParams(
        use_tc_tiling_on_sc=True, needs_layout_passes=False
    )
    scratch = pl.kernel(
        _relayout_body,
        mesh=mesh,
        compiler_params=params,
        out_type=jax.ShapeDtypeStruct((_SR, 128), jnp.float32),
        scratch_types=[
            pltpu.VMEM((_EMB, _CHUNK), jnp.float32),
            pltpu.VMEM((_EMB, _CHUNK), jnp.float32),
            pltpu.VMEM((_CHUNK // 4, 128), jnp.float32),
            pltpu.VMEM((_CHUNK // 4, 128), jnp.float32),
            pltpu.VMEM((_EMB, _TAIL), jnp.float32),
            pltpu.SemaphoreType.DMA,
            pltpu.SemaphoreType.DMA,
            pltpu.SemaphoreType.DMA,
            pltpu.SemaphoreType.DMA,
        ],
    )(wt_t, tail_t)
    out_t = pl.kernel(
        _lookup_body,
        mesh=mesh,
        compiler_params=params,
        out_type=jax.ShapeDtypeStruct((_L, _EMB + _PDIM, _B), jnp.float32),
        scratch_types=[
            pltpu.VMEM((_PDIM, 512), jnp.float32),
            pltpu.VMEM((8, 128), jnp.int32),
            pltpu.VMEM((8, 128), jnp.int32),
            pltpu.VMEM((8, 128), jnp.int32),
            pltpu.VMEM((128, 128), jnp.float32),
            pltpu.VMEM((128, 128), jnp.float32),
            pltpu.VMEM((_EMB + _PDIM, 128), jnp.float32),
            pltpu.VMEM((_EMB + _PDIM, 128), jnp.float32),
            pltpu.SemaphoreType.DMA,
            pltpu.SemaphoreType.DMA,
            pltpu.SemaphoreType.DMA,
            pltpu.SemaphoreType.DMA,
        ],
    )(scratch, ids_t, pos_t)
    return out_t


def kernel(input_ids, word_table, pos_table):
    ids_t = input_ids.T
    wt_t = word_table.T
    tail_t = word_table[_NFULL * _CHUNK:, :].T
    pos_t = pos_table.T
    out_t = _emb_fused(ids_t, wt_t, tail_t, pos_t)
    return jnp.transpose(out_t, (2, 0, 1))
